# Initial kernel scaffold; baseline (speedup 1.0000x reference)
import jax
import jax.numpy as jnp
from jax.experimental import pallas as pl


def kernel(x, pos, edge_index, batch, W1_l, W1_r, b1, W2_l, W2_r, b2, W_lin, b_lin):
    raise NotImplementedError("write your pallas kernel here")


# speedup vs baseline: 14.1443x; 14.1443x over previous
import functools

import jax
import jax.numpy as jnp
from jax import lax
from jax.experimental import pallas as pl
from jax.experimental.pallas import tpu as pltpu
from jax.experimental.pallas import tpu_sc as plsc

N = 100000
E = 3200000
G = 512
HID = 64

NC, NS, LANES = 2, 16, 16
NW = NC * NS

BLK = 2048
NB = 49
N_ACC = NB * BLK
N_SP = 100016
RPT = N_SP // NS
ZR = 329
N_TAIL = N_ACC - N_SP

NSUB = 8
CHUNK = NSUB * 128
E_W = N_ACC
E_PAD = NW * E_W
RW = E_W // 128
NCHUNK = RW // NSUB

DUMMY_DST = N


def _make_agg(T):
    mesh = plsc.VectorSubcoreMesh(
        core_axis_name="c", subcore_axis_name="s",
        num_cores=NC, num_subcores=NS)

    @functools.partial(
        pl.kernel,
        out_type=jax.ShapeDtypeStruct((NC, T, N_ACC, 16), jnp.float32),
        mesh=mesh,
        scratch_types=[
            pltpu.VMEM((NSUB, 128), jnp.int32),
            pltpu.VMEM((NSUB, 128), jnp.int32),
            pltpu.VMEM((NSUB, 128, 16), jnp.float32),
            pltpu.VMEM((ZR, 16), jnp.float32),
            pltpu.VMEM_SHARED((N_SP, 16), jnp.float32),
            pltpu.SemaphoreType.DMA,
        ],
        compiler_params=pltpu.CompilerParams(use_tc_tiling_on_sc=False),
    )
    def agg(src_hbm, dst_hbm, *rest):
        tables = rest[:T]
        out = rest[T]
        src_v, dst_v, rows_v, zbuf, acc, sem = rest[T + 1:]

        cid = lax.axis_index("c")
        sid = lax.axis_index("s")
        wid = cid * NS + sid
        my0 = sid * RPT

        def zb(z, carry):
            zbuf[z] = jnp.zeros((16,), jnp.float32)
            return carry
        lax.fori_loop(0, ZR, zb, 0)

        def zero_acc():
            for z in range(RPT // ZR):
                pltpu.sync_copy(zbuf, acc.at[pl.ds(my0 + z * ZR, ZR)])

        zero_acc()
        plsc.subcore_barrier()

        for t in range(T):
            table = tables[t]

            def chunk_body(i, carry):
                row0 = wid * RW + i * NSUB
                pltpu.sync_copy(src_hbm.at[pl.ds(row0, NSUB)], src_v)
                pltpu.sync_copy(dst_hbm.at[pl.ds(row0, NSUB)], dst_v)
                descs = [
                    pltpu.async_copy(table.at[src_v.at[j]], rows_v.at[j], sem)
                    for j in range(NSUB)
                ]
                for d in descs:
                    d.wait()
                for j in range(NSUB):
                    pltpu.sync_copy(rows_v.at[j], acc.at[dst_v.at[j]],
                                    add=True)
                return carry

            lax.fori_loop(0, NCHUNK, chunk_body, 0)
            plsc.subcore_barrier()
            pltpu.sync_copy(acc.at[pl.ds(my0, RPT)],
                            out.at[cid, t, pl.ds(my0, RPT)])

            @pl.when(sid == 0)
            def _():
                for off in range(0, N_TAIL, ZR):
                    w = min(ZR, N_TAIL - off)
                    pltpu.sync_copy(zbuf.at[pl.ds(0, w)],
                                    out.at[cid, t, pl.ds(N_SP + off, w)])

            if t + 1 < T:
                zero_acc()
            plsc.subcore_barrier()

    return agg


_agg1 = _make_agg(1)
_agg2 = _make_agg(4)


def _tc1_body(acc_ref, h0_ref, wl_ref, wr_ref,
              c0_ref, c1_ref, c2_ref, c3_ref, cnt_ref):
    agg = acc_ref[0, 0] + acc_ref[1, 0]
    cnt = jnp.maximum(agg[:, 14:15], 1.0)
    mean = agg / cnt
    h0 = h0_ref[...]
    h1 = jnp.maximum(
        jnp.dot(mean, wl_ref[...], preferred_element_type=jnp.float32)
        + jnp.dot(h0, wr_ref[...], preferred_element_type=jnp.float32),
        0.0)
    c0_ref[...] = h1[:, 0:16]
    c1_ref[...] = h1[:, 16:32]
    c2_ref[...] = h1[:, 32:48]
    c3_ref[...] = h1[:, 48:64]
    cnt_ref[...] = jnp.broadcast_to(cnt, (BLK, 8))


def _tc1(acc1, h0p, w1l, w1r):
    cspec = pl.BlockSpec((BLK, 16), lambda i: (i, 0))
    return pl.pallas_call(
        _tc1_body,
        grid=(NB,),
        in_specs=[
            pl.BlockSpec((NC, 1, BLK, 16), lambda i: (0, 0, i, 0)),
            pl.BlockSpec((BLK, 16), lambda i: (i, 0)),
            pl.BlockSpec((16, HID), lambda i: (0, 0)),
            pl.BlockSpec((16, HID), lambda i: (0, 0)),
        ],
        out_specs=[cspec, cspec, cspec, cspec,
                   pl.BlockSpec((BLK, 8), lambda i: (i, 0))],
        out_shape=[jax.ShapeDtypeStruct((N_ACC, 16), jnp.float32)] * 4
        + [jax.ShapeDtypeStruct((N_ACC, 8), jnp.float32)],
    )(acc1, h0p, w1l, w1r)


def _tc2_body(acc_ref, h10_ref, h11_ref, h12_ref, h13_ref, cnt_ref,
              w2l_ref, w2r_ref, b2_ref,
              bt_ref, wlin_ref, blin_ref, out_ref, g_acc, c_acc):
    i = pl.program_id(0)

    @pl.when(i == 0)
    def _():
        g_acc[...] = jnp.zeros((G, 8), jnp.float32)
        c_acc[...] = jnp.zeros((G, 8), jnp.float32)

    s = acc_ref[0] + acc_ref[1]
    mean2 = jnp.concatenate([s[0], s[1], s[2], s[3]], axis=1)
    mean2 = mean2 / cnt_ref[:, 0:1]
    h1 = jnp.concatenate([h10_ref[...], h11_ref[...], h12_ref[...],
                          h13_ref[...]], axis=1)
    z = (jnp.dot(mean2, w2l_ref[...], preferred_element_type=jnp.float32)
         + jnp.dot(h1, w2r_ref[...], preferred_element_type=jnp.float32)
         + b2_ref[0:1, :])
    h2 = jnp.maximum(z, 0.0)
    y = jnp.dot(h2, wlin_ref[...], preferred_element_type=jnp.float32)
    bt = bt_ref[0]
    oh = (lax.broadcasted_iota(jnp.int32, (G, BLK), 0) == bt)
    ohf = oh.astype(jnp.float32)
    g_acc[...] += jnp.dot(ohf, y, preferred_element_type=jnp.float32)
    c_acc[...] += jnp.broadcast_to(
        jnp.sum(ohf, axis=1, keepdims=True), (G, 8))

    @pl.when(i == NB - 1)
    def _():
        out_ref[...] = (g_acc[...] / jnp.maximum(c_acc[...], 1.0)
                        + blin_ref[0:1, 0:1])


def _tc2(acc2, h1c0, h1c1, h1c2, h1c3, cntc, w2l, w2r, b2, bt3, wlin, blin):
    cspec = pl.BlockSpec((BLK, 16), lambda i: (i, 0))
    return pl.pallas_call(
        _tc2_body,
        grid=(NB,),
        in_specs=[
            pl.BlockSpec((NC, 4, BLK, 16), lambda i: (0, 0, i, 0)),
            cspec, cspec, cspec, cspec,
            pl.BlockSpec((BLK, 8), lambda i: (i, 0)),
            pl.BlockSpec((HID, HID), lambda i: (0, 0)),
            pl.BlockSpec((HID, HID), lambda i: (0, 0)),
            pl.BlockSpec((8, HID), lambda i: (0, 0)),
            pl.BlockSpec((1, 1, BLK), lambda i: (i, 0, 0)),
            pl.BlockSpec((HID, 8), lambda i: (0, 0)),
            pl.BlockSpec((8, 8), lambda i: (0, 0)),
        ],
        out_specs=pl.BlockSpec((G, 8), lambda i: (0, 0)),
        out_shape=jax.ShapeDtypeStruct((G, 8), jnp.float32),
        scratch_shapes=[
            pltpu.VMEM((G, 8), jnp.float32),
            pltpu.VMEM((G, 8), jnp.float32),
        ],
    )(acc2, h1c0, h1c1, h1c2, h1c3, cntc, w2l, w2r, b2, bt3, wlin, blin)


def kernel(x, pos, edge_index, batch, W1_l, W1_r, b1, W2_l, W2_r, b2,
           W_lin, b_lin):
    f32 = jnp.float32
    src = edge_index[0].astype(jnp.int32)
    dst = edge_index[1].astype(jnp.int32)
    src_p = jnp.concatenate(
        [src, jnp.zeros((E_PAD - E,), jnp.int32)]).reshape(E_PAD // 128, 128)
    dst_p = jnp.concatenate(
        [dst, jnp.full((E_PAD - E,), DUMMY_DST, jnp.int32)]
    ).reshape(E_PAD // 128, 128)

    ones = jnp.ones((N, 1), f32)
    zcol = jnp.zeros((N, 1), f32)
    h0p = jnp.concatenate([x, pos, ones, zcol], axis=1)
    h0p = jnp.pad(h0p, ((0, N_ACC - N), (0, 0)))

    w1l = jnp.zeros((16, HID), f32).at[:14].set(W1_l)
    w1r = jnp.zeros((16, HID), f32).at[:14].set(W1_r).at[14].set(b1)
    b2_8 = jnp.broadcast_to(b2[None, :], (8, HID))
    wlin8 = jnp.zeros((HID, 8), f32).at[:, 0:1].set(W_lin)
    blin8 = jnp.broadcast_to(b_lin.reshape(1, 1), (8, 8)).astype(f32)
    bt3 = jnp.pad(batch.astype(jnp.int32), (0, N_ACC - N),
                  constant_values=G).reshape(NB, 1, BLK)

    acc1 = _agg1(src_p, dst_p, h0p)
    h1c0, h1c1, h1c2, h1c3, cntc = _tc1(acc1, h0p, w1l, w1r)

    acc2 = _agg2(src_p, dst_p, h1c0, h1c1, h1c2, h1c3)
    out8 = _tc2(acc2, h1c0, h1c1, h1c2, h1c3, cntc,
                W2_l, W2_r, b2_8, bt3, wlin8, blin8)
    return out8[:, 0:1]


# speedup vs baseline: 19.5817x; 1.3844x over previous
import functools

import jax
import jax.numpy as jnp
from jax import lax
from jax.experimental import pallas as pl
from jax.experimental.pallas import tpu as pltpu
from jax.experimental.pallas import tpu_sc as plsc

N = 100000
E = 3200000
G = 512
HID = 64

NC, NS, LANES = 2, 16, 16
NW = NC * NS

BLK = 2048
NB = 49
N_ACC = NB * BLK
N_SP = 100016
RPT = N_SP // NS
N_TAIL = N_ACC - N_SP

CH = 784
NCH = 128
E_W = CH * NCH
E_PAD = NW * E_W

DUMMY_DST = N


def _make_agg(T):
    mesh = plsc.VectorSubcoreMesh(
        core_axis_name="c", subcore_axis_name="s",
        num_cores=NC, num_subcores=NS)

    @functools.partial(
        pl.kernel,
        out_type=jax.ShapeDtypeStruct((NC, T, N_ACC, 16), jnp.float32),
        mesh=mesh,
        scratch_types=[
            pltpu.VMEM((CH,), jnp.int32),
            pltpu.VMEM((CH,), jnp.int32),
            pltpu.VMEM((CH,), jnp.int32),
            pltpu.VMEM((CH,), jnp.int32),
            pltpu.VMEM((CH, 16), jnp.float32),
            pltpu.VMEM((CH, 16), jnp.float32),
            pltpu.VMEM_SHARED((N_SP, 16), jnp.float32),
            pltpu.SemaphoreType.DMA,
            pltpu.SemaphoreType.DMA,
            pltpu.SemaphoreType.DMA,
        ],
        compiler_params=pltpu.CompilerParams(use_tc_tiling_on_sc=False),
    )
    def agg(src_hbm, dst_hbm, zeros_hbm, *rest):
        tables = rest[:T]
        out = rest[T]
        (src0, src1, dst0, dst1, rows0, rows1,
         acc, isem, gsem, ssem) = rest[T + 1:]
        bufs = [(src0, dst0, rows0), (src1, dst1, rows1)]

        cid = lax.axis_index("c")
        sid = lax.axis_index("s")
        wid = cid * NS + sid
        my0 = sid * RPT
        base = wid * E_W

        def fire_idx(i, sv, dv):
            pltpu.async_copy(src_hbm.at[pl.ds(base + i * CH, CH)], sv, isem)
            pltpu.async_copy(dst_hbm.at[pl.ds(base + i * CH, CH)], dv, isem)

        def wait_idx(sv, dv):
            pltpu.make_async_copy(src_hbm.at[pl.ds(base, CH)], sv,
                                  isem).wait()
            pltpu.make_async_copy(dst_hbm.at[pl.ds(base, CH)], dv,
                                  isem).wait()

        def drain_scatter(dv, rv):
            pltpu.make_async_copy(rv, acc.at[dv], ssem).wait()

        pltpu.sync_copy(zeros_hbm, acc.at[pl.ds(my0, RPT)])
        plsc.subcore_barrier()

        for t in range(T):
            table = tables[t]

            fire_idx(0, src0, dst0)

            def pair_body(k, carry):
                for b in range(2):
                    i = k * 2 + b
                    sv, dv, rv = bufs[b]
                    svn, dvn, rvn = bufs[1 - b]
                    wait_idx(sv, dv)
                    gd = pltpu.async_copy(table.at[sv], rv, gsem)

                    @pl.when(i > 0)
                    def _():
                        drain_scatter(dvn, rvn)

                    @pl.when(i + 1 < NCH)
                    def _():
                        fire_idx(i + 1, svn, dvn)

                    gd.wait()
                    pltpu.async_copy(rv, acc.at[dv], ssem, add=True)
                return carry

            lax.fori_loop(0, NCH // 2, pair_body, 0)
            drain_scatter(dst1, rows1)
            plsc.subcore_barrier()
            pltpu.sync_copy(acc.at[pl.ds(my0, RPT)],
                            out.at[cid, t, pl.ds(my0, RPT)])

            @pl.when(sid == 0)
            def _():
                pltpu.sync_copy(zeros_hbm.at[pl.ds(0, N_TAIL)],
                                out.at[cid, t, pl.ds(N_SP, N_TAIL)])

            if t + 1 < T:
                pltpu.sync_copy(zeros_hbm, acc.at[pl.ds(my0, RPT)])
            plsc.subcore_barrier()

    return agg


_agg1 = _make_agg(1)
_agg2 = _make_agg(4)


def _dot(a, b):
    return jnp.dot(a, b, preferred_element_type=jnp.float32)


def _dot_hilo(a_exact, b):
    ab = a_exact.astype(jnp.bfloat16)
    bh = b.astype(jnp.bfloat16)
    bl = (b - bh.astype(jnp.float32)).astype(jnp.bfloat16)
    return (jnp.dot(ab, bh, preferred_element_type=jnp.float32)
            + jnp.dot(ab, bl, preferred_element_type=jnp.float32))


def _tc1_body(acc_ref, h0_ref, wl_ref, wr_ref,
              c0_ref, c1_ref, c2_ref, c3_ref, cnt_ref):
    agg = acc_ref[0, 0] + acc_ref[1, 0]
    cnt = jnp.maximum(agg[:, 14:15], 1.0)
    mean = agg / cnt
    h0 = h0_ref[...]
    h1 = jnp.maximum(
        _dot(mean, wl_ref[...]) + _dot(h0, wr_ref[...]),
        0.0)
    c0_ref[...] = h1[:, 0:16]
    c1_ref[...] = h1[:, 16:32]
    c2_ref[...] = h1[:, 32:48]
    c3_ref[...] = h1[:, 48:64]
    cnt_ref[...] = jnp.broadcast_to(cnt, (BLK, 8))


def _tc1(acc1, h0p, w1l, w1r):
    cspec = pl.BlockSpec((BLK, 16), lambda i: (i, 0))
    return pl.pallas_call(
        _tc1_body,
        grid=(NB,),
        in_specs=[
            pl.BlockSpec((NC, 1, BLK, 16), lambda i: (0, 0, i, 0)),
            pl.BlockSpec((BLK, 16), lambda i: (i, 0)),
            pl.BlockSpec((16, HID), lambda i: (0, 0)),
            pl.BlockSpec((16, HID), lambda i: (0, 0)),
        ],
        out_specs=[cspec, cspec, cspec, cspec,
                   pl.BlockSpec((BLK, 8), lambda i: (i, 0))],
        out_shape=[jax.ShapeDtypeStruct((N_ACC, 16), jnp.float32)] * 4
        + [jax.ShapeDtypeStruct((N_ACC, 8), jnp.float32)],
    )(acc1, h0p, w1l, w1r)


def _tc2_body(acc_ref, h10_ref, h11_ref, h12_ref, h13_ref, cnt_ref,
              w2l_ref, w2r_ref, b2_ref,
              bt_ref, wlin_ref, blin_ref, out_ref, g_acc, c_acc):
    i = pl.program_id(0)

    @pl.when(i == 0)
    def _():
        g_acc[...] = jnp.zeros((G, HID), jnp.float32)
        c_acc[...] = jnp.zeros((G, 8), jnp.float32)

    s = acc_ref[0] + acc_ref[1]
    mean2 = jnp.concatenate([s[0], s[1], s[2], s[3]], axis=1)
    mean2 = mean2 / cnt_ref[:, 0:1]
    h1 = jnp.concatenate([h10_ref[...], h11_ref[...], h12_ref[...],
                          h13_ref[...]], axis=1)
    z = (_dot(mean2, w2l_ref[...]) + _dot(h1, w2r_ref[...])
         + b2_ref[0:1, :])
    h2 = jnp.maximum(z, 0.0)
    bt = bt_ref[0]
    oh = (lax.broadcasted_iota(jnp.int32, (G, BLK), 0) == bt)
    ohf = oh.astype(jnp.float32)
    g_acc[...] += _dot_hilo(ohf, h2)
    c_acc[...] += jnp.broadcast_to(
        jnp.sum(ohf, axis=1, keepdims=True), (G, 8))

    @pl.when(i == NB - 1)
    def _():
        g = g_acc[...] / jnp.maximum(c_acc[:, 0:1], 1.0)
        out_ref[...] = _dot(g, wlin_ref[...]) + blin_ref[0:1, 0:1]


def _tc2(acc2, h1c0, h1c1, h1c2, h1c3, cntc, w2l, w2r, b2, bt3, wlin, blin):
    cspec = pl.BlockSpec((BLK, 16), lambda i: (i, 0))
    return pl.pallas_call(
        _tc2_body,
        grid=(NB,),
        in_specs=[
            pl.BlockSpec((NC, 4, BLK, 16), lambda i: (0, 0, i, 0)),
            cspec, cspec, cspec, cspec,
            pl.BlockSpec((BLK, 8), lambda i: (i, 0)),
            pl.BlockSpec((HID, HID), lambda i: (0, 0)),
            pl.BlockSpec((HID, HID), lambda i: (0, 0)),
            pl.BlockSpec((8, HID), lambda i: (0, 0)),
            pl.BlockSpec((1, 1, BLK), lambda i: (i, 0, 0)),
            pl.BlockSpec((HID, 8), lambda i: (0, 0)),
            pl.BlockSpec((8, 8), lambda i: (0, 0)),
        ],
        out_specs=pl.BlockSpec((G, 8), lambda i: (0, 0)),
        out_shape=jax.ShapeDtypeStruct((G, 8), jnp.float32),
        scratch_shapes=[
            pltpu.VMEM((G, HID), jnp.float32),
            pltpu.VMEM((G, 8), jnp.float32),
        ],
    )(acc2, h1c0, h1c1, h1c2, h1c3, cntc, w2l, w2r, b2, bt3, wlin, blin)


def kernel(x, pos, edge_index, batch, W1_l, W1_r, b1, W2_l, W2_r, b2,
           W_lin, b_lin):
    f32 = jnp.float32
    src = edge_index[0].astype(jnp.int32)
    dst = edge_index[1].astype(jnp.int32)
    src_p = jnp.concatenate([src, jnp.zeros((E_PAD - E,), jnp.int32)])
    dst_p = jnp.concatenate(
        [dst, jnp.full((E_PAD - E,), DUMMY_DST, jnp.int32)])
    zrows = jnp.zeros((RPT, 16), f32)

    ones = jnp.ones((N, 1), f32)
    zcol = jnp.zeros((N, 1), f32)
    h0p = jnp.concatenate([x, pos, ones, zcol], axis=1)
    h0p = jnp.pad(h0p, ((0, N_ACC - N), (0, 0)))

    w1l = jnp.zeros((16, HID), f32).at[:14].set(W1_l)
    w1r = jnp.zeros((16, HID), f32).at[:14].set(W1_r).at[14].set(b1)
    b2_8 = jnp.broadcast_to(b2[None, :], (8, HID))
    wlin8 = jnp.zeros((HID, 8), f32).at[:, 0:1].set(W_lin)
    blin8 = jnp.broadcast_to(b_lin.reshape(1, 1), (8, 8)).astype(f32)
    bt3 = jnp.pad(batch.astype(jnp.int32), (0, N_ACC - N),
                  constant_values=G).reshape(NB, 1, BLK)

    acc1 = _agg1(src_p, dst_p, zrows, h0p)
    h1c0, h1c1, h1c2, h1c3, cntc = _tc1(acc1, h0p, w1l, w1r)

    acc2 = _agg2(src_p, dst_p, zrows, h1c0, h1c1, h1c2, h1c3)
    out8 = _tc2(acc2, h1c0, h1c1, h1c2, h1c3, cntc,
                W2_l, W2_r, b2_8, bt3, wlin8, blin8)
    return out8[:, 0:1]
